# Initial kernel scaffold; baseline (speedup 1.0000x reference)
import jax
import jax.numpy as jnp
from jax.experimental import pallas as pl


def kernel(x, edge_index, W1, b1, W2, b2, W3, b3):
    raise NotImplementedError("write your pallas kernel here")


# speedup vs baseline: 9.0407x; 9.0407x over previous
import dataclasses
import functools

import jax
import jax.numpy as jnp
from jax import lax
from jax.experimental import pallas as pl
from jax.experimental.pallas import tpu as pltpu
from jax.experimental.pallas import tpu_sc as plsc

N = 10000
E = 320000
D = 128
NC, NS = 2, 16
C = 128
NP = 10240
RPT = NP // NS
EC = 161792
ET = EC // NS
K = ET // C
E_PAD = EC * NC
BM = 256

_mesh = plsc.VectorSubcoreMesh(
    core_axis_name="c", subcore_axis_name="s", num_cores=NC, num_subcores=NS
)

_cp = pltpu.CompilerParams()
if "needs_layout_passes" in pltpu.CompilerParams.__dataclass_fields__:
    _cp = dataclasses.replace(_cp, needs_layout_passes=False)


@functools.partial(
    pl.kernel,
    out_type=jax.ShapeDtypeStruct((NC * NS, NP), jnp.float32),
    mesh=_mesh,
    compiler_params=_cp,
    scratch_types=[
        pltpu.VMEM((C,), jnp.int32),
        pltpu.VMEM((NP,), jnp.float32),
    ],
)
def _sc_degree(dst_hbm, zeros_hbm, out_hbm, dbuf, cnt_v):
    c = lax.axis_index("c")
    s = lax.axis_index("s")
    pltpu.sync_copy(zeros_hbm, cnt_v)
    base = c * EC + s * ET
    ones = jnp.full((16,), 1.0, jnp.float32)

    @pl.loop(0, K)
    def _(k):
        pltpu.sync_copy(dst_hbm.at[pl.ds(base + k * C, C)], dbuf)

        @pl.loop(0, C // 16)
        def _(j):
            idx = dbuf[pl.ds(j * 16, 16)]
            plsc.addupdate_scatter(cnt_v, [idx], ones)

    pltpu.sync_copy(cnt_v, out_hbm.at[c * NS + s])


@functools.partial(
    pl.kernel,
    out_type=jax.ShapeDtypeStruct((NC, NP, D), jnp.float32),
    mesh=_mesh,
    scratch_types=[
        pltpu.VMEM((C,), jnp.int32),
        pltpu.VMEM((C,), jnp.int32),
        pltpu.VMEM((C, D), jnp.float32),
        pltpu.VMEM_SHARED((NP, D), jnp.float32),
        pltpu.SemaphoreType.DMA,
    ],
)
def _sc_aggregate(h_hbm, src_hbm, dst_hbm, out_hbm, sbuf, dbuf, rows_v, acc_sh, sem):
    c = lax.axis_index("c")
    s = lax.axis_index("s")
    rows = pl.ds(s * RPT, RPT)
    pltpu.sync_copy(h_hbm.at[rows], acc_sh.at[rows])
    plsc.subcore_barrier()
    base = c * EC + s * ET

    @pl.loop(0, K)
    def _(k):
        off = base + k * C
        pltpu.sync_copy(src_hbm.at[pl.ds(off, C)], sbuf)
        pltpu.sync_copy(dst_hbm.at[pl.ds(off, C)], dbuf)
        pltpu.async_copy(h_hbm.at[sbuf], rows_v, sem).wait()
        pltpu.sync_copy(rows_v, acc_sh.at[dbuf], add=True)

    plsc.subcore_barrier()
    pltpu.sync_copy(acc_sh.at[rows], out_hbm.at[c, rows])


def _mm(x, W):
    def body(x_ref, w_ref, o_ref):
        o_ref[...] = jax.lax.dot(
            x_ref[...], w_ref[...], precision=jax.lax.Precision.HIGHEST
        )

    return pl.pallas_call(
        body,
        grid=(NP // BM,),
        in_specs=[
            pl.BlockSpec((BM, D), lambda i: (i, 0)),
            pl.BlockSpec((D, D), lambda i: (0, 0)),
        ],
        out_specs=pl.BlockSpec((BM, D), lambda i: (i, 0)),
        out_shape=jax.ShapeDtypeStruct((NP, D), jnp.float32),
    )(x, W)


def _scale(h, degt):
    def body(h_ref, g_ref, hp_ref, dis_ref):
        deg = jnp.sum(g_ref[...], axis=1, keepdims=True) + 1.0
        dis = jax.lax.rsqrt(deg)
        hp_ref[...] = h_ref[...] * dis
        dis_ref[...] = dis

    return pl.pallas_call(
        body,
        grid=(NP // BM,),
        in_specs=[
            pl.BlockSpec((BM, D), lambda i: (i, 0)),
            pl.BlockSpec((BM, NC * NS), lambda i: (i, 0)),
        ],
        out_specs=[
            pl.BlockSpec((BM, D), lambda i: (i, 0)),
            pl.BlockSpec((BM, 1), lambda i: (i, 0)),
        ],
        out_shape=[
            jax.ShapeDtypeStruct((NP, D), jnp.float32),
            jax.ShapeDtypeStruct((NP, 1), jnp.float32),
        ],
    )(h, degt)


def _combine(acc2, hp, dis, b, W):
    def body(a_ref, hp_ref, dis_ref, b_ref, w_ref, o_ref):
        ssum = a_ref[0] + a_ref[1] - hp_ref[...]
        o = dis_ref[...] * ssum + b_ref[...]
        a = jnp.maximum(o, 0.0)
        o_ref[...] = (
            jax.lax.dot(a, w_ref[...], precision=jax.lax.Precision.HIGHEST)
            * dis_ref[...]
        )

    return pl.pallas_call(
        body,
        grid=(NP // BM,),
        in_specs=[
            pl.BlockSpec((NC, BM, D), lambda i: (0, i, 0)),
            pl.BlockSpec((BM, D), lambda i: (i, 0)),
            pl.BlockSpec((BM, 1), lambda i: (i, 0)),
            pl.BlockSpec((1, D), lambda i: (0, 0)),
            pl.BlockSpec((D, D), lambda i: (0, 0)),
        ],
        out_specs=pl.BlockSpec((BM, D), lambda i: (i, 0)),
        out_shape=jax.ShapeDtypeStruct((NP, D), jnp.float32),
    )(acc2, hp, dis, b, W)


def _final(acc2, hp, dis, b):
    def body(a_ref, hp_ref, dis_ref, b_ref, o_ref):
        ssum = a_ref[0] + a_ref[1] - hp_ref[...]
        o_ref[...] = dis_ref[...] * ssum + b_ref[...]

    return pl.pallas_call(
        body,
        grid=(NP // BM,),
        in_specs=[
            pl.BlockSpec((NC, BM, D), lambda i: (0, i, 0)),
            pl.BlockSpec((BM, D), lambda i: (i, 0)),
            pl.BlockSpec((BM, 1), lambda i: (i, 0)),
            pl.BlockSpec((1, D), lambda i: (0, 0)),
        ],
        out_specs=pl.BlockSpec((BM, D), lambda i: (i, 0)),
        out_shape=jax.ShapeDtypeStruct((NP, D), jnp.float32),
    )(acc2, hp, dis, b)


@jax.jit
def kernel(x, edge_index, W1, b1, W2, b2, W3, b3):
    src = edge_index[0].astype(jnp.int32)
    dst = edge_index[1].astype(jnp.int32)
    pad_e = jnp.full((E_PAD - E,), N, jnp.int32)
    src_p = jnp.concatenate([src, pad_e])
    dst_p = jnp.concatenate([dst, pad_e])
    x_p = jnp.zeros((NP, D), jnp.float32).at[:N].set(x)
    zeros1 = jnp.zeros((NP,), jnp.float32)

    degp = _sc_degree(dst_p, zeros1)
    h1 = _mm(x_p, W1)
    h1p, dis = _scale(h1, degp.T)
    acc1 = _sc_aggregate(h1p, src_p, dst_p)
    h2p = _combine(acc1, h1p, dis, b1.reshape(1, D), W2)
    acc2 = _sc_aggregate(h2p, src_p, dst_p)
    h3p = _combine(acc2, h2p, dis, b2.reshape(1, D), W3)
    acc3 = _sc_aggregate(h3p, src_p, dst_p)
    out = _final(acc3, h3p, dis, b3.reshape(1, D))
    return out[:N]
